# baseline (speedup 1.0000x reference)
import functools

import jax
import jax.numpy as jnp
from jax import lax
from jax.experimental import pallas as pl
from jax.experimental.pallas import tpu as pltpu
from jax.experimental.pallas import tpu_sc as plsc

_D = 64
_L = 16
_HIST = 200
_RPT = 16
_NHT = 13

_mesh = plsc.VectorSubcoreMesh(core_axis_name="c", subcore_axis_name="s")


@functools.partial(
    pl.kernel,
    out_type=jax.ShapeDtypeStruct((_L,), jnp.float32),
    mesh=_mesh,
    scratch_types=[
        pltpu.VMEM((_L,), jnp.int32),
        pltpu.VMEM((_RPT, _D), jnp.float32),
        pltpu.VMEM((_D,), jnp.float32),
        pltpu.VMEM((24, _D), jnp.float32),
        pltpu.VMEM((_L,), jnp.float32),
        pltpu.VMEM((_L,), jnp.float32),
        pltpu.VMEM_SHARED((24, _D), jnp.float32),
        pltpu.SemaphoreType.DMA,
    ],
    compiler_params=pltpu.CompilerParams(needs_layout_passes=False,
                                         use_tc_tiling_on_sc=False),
)
def _sc_bpr(uidx_hbm, iidx_hbm, utab_hbm, itab_hbm, out_hbm,
            idx_v, rows_v, part_v, all_v, out_v, red_v, shared, sem):
    cid = lax.axis_index("c")
    sid = lax.axis_index("s")
    on0 = cid == 0

    @pl.when(jnp.logical_and(on0, sid < _NHT))
    def _():
        base = sid * _RPT
        pltpu.sync_copy(iidx_hbm.at[pl.ds(base, _RPT)], idx_v)
        pltpu.async_copy(itab_hbm.at[idx_v], rows_v, sem).wait()
        for j in range(_D // _L):
            acc = jnp.zeros((_L,), jnp.float32)
            for i in range(_RPT):
                w = jnp.where(base + i < _HIST, jnp.float32(1.0),
                              jnp.float32(0.0))
                acc = acc + rows_v[i, _L * j:_L * (j + 1)] * w
            part_v[pl.ds(_L * j, _L)] = acc
        pltpu.sync_copy(part_v, shared.at[sid])

    @pl.when(jnp.logical_and(on0, sid == _NHT - 1))
    def _():
        pltpu.sync_copy(rows_v.at[8], shared.at[17])
        pltpu.sync_copy(rows_v.at[9], shared.at[18])

    @pl.when(jnp.logical_and(on0, sid == _NHT))
    def _():
        pltpu.sync_copy(uidx_hbm, idx_v)
        pltpu.async_copy(utab_hbm.at[idx_v], rows_v, sem).wait()
        pltpu.sync_copy(rows_v.at[0], shared.at[16])

    plsc.subcore_barrier()

    @pl.when(jnp.logical_and(on0, sid == 0))
    def _():
        pltpu.sync_copy(shared, all_v)
        accp = jnp.zeros((_L,), jnp.float32)
        accn = jnp.zeros((_L,), jnp.float32)
        for j in range(_D // _L):
            sl = slice(_L * j, _L * (j + 1))
            h = all_v[0, sl]
            for t in range(1, _NHT):
                h = h + all_v[t, sl]
            s = all_v[16, sl] + h / jnp.float32(_HIST)
            accp = accp + s * all_v[17, sl]
            accn = accn + s * all_v[18, sl]
        lane = lax.iota(jnp.int32, _L)

        def lane_sum(v):
            for sh in (8, 4, 2, 1):
                red_v[...] = v
                v = v + plsc.load_gather(red_v, [lane ^ sh])
            return v

        ps = lane_sum(accp)
        ns = lane_sum(accn)
        out_v[...] = jnp.where(lane == 0, ps,
                               jnp.where(lane == 1, ns, jnp.float32(0.0)))
        pltpu.sync_copy(out_v, out_hbm)


def kernel(user_id, pos_item_id, neg_item_id, item_history, user_table,
           item_table):
    iidx = jnp.concatenate([
        item_history.astype(jnp.int32),
        jnp.asarray(pos_item_id, jnp.int32)[None],
        jnp.asarray(neg_item_id, jnp.int32)[None],
        jnp.zeros((6,), jnp.int32),
    ])
    uidx = jnp.zeros((_L,), jnp.int32).at[0].set(
        jnp.asarray(user_id, jnp.int32))
    out = _sc_bpr(uidx, iidx, user_table, item_table)
    return (out[0], out[1])
